# baseline (speedup 1.0000x reference)
import jax
import jax.numpy as jnp
from jax.experimental import pallas as pl
from jax.experimental.pallas import tpu as pltpu

_N = 512
_DE = 64
_DN = 128
_DH = 512
_TJ = 128


def _body(a_ref, b_ref, wet_ref, wnt_ref, bn_ref, w1a_ref, w1b_ref, b1_ref,
          w2_ref, b2_ref, o_ref, p_s, q_s):
    f32 = jnp.float32
    bf = jnp.bfloat16
    i = pl.program_id(0)

    @pl.when(i == 0)
    def _prep():
        wct = wet_ref[...]
        wnt_b = wnt_ref[...].astype(bf)

        def node_t(x):
            cs = jnp.sum(x, axis=0, keepdims=True)
            ca = jnp.sum(jnp.abs(x), axis=0, keepdims=True)
            aggt = 0.505 * wct * cs + 0.495 * jnp.abs(wct) * ca
            z = jnp.dot(wnt_b, aggt.astype(bf),
                        preferred_element_type=f32) + bn_ref[...]
            return jnp.maximum(z, 0.01 * z)

        nat = node_t(a_ref[...]).astype(bf)
        nbt = node_t(b_ref[...]).astype(bf)
        dc = (((0,), (0,)), ((), ()))
        p_s[...] = jax.lax.dot_general(nat, w1a_ref[...].astype(bf), dc,
                                       preferred_element_type=f32)
        q_s[...] = jax.lax.dot_general(nbt, w1b_ref[...].astype(bf), dc,
                                       preferred_element_type=f32) \
            + b1_ref[...]

    p = p_s[...]
    w2c = w2_ref[...].astype(bf)
    b2 = b2_ref[...]
    for t in range(_TJ):
        qrow = q_s[pl.ds(i * _TJ + t, 1), :]
        s = (p + qrow).astype(bf)
        lb = jnp.maximum(s, bf(0.01) * s)
        c = jax.lax.dot_general(lb, w2c, (((1,), (0,)), ((), ())),
                                preferred_element_type=f32)
        o_ref[:, t:t + 1] = c + b2


def kernel(A, B, linear_costs, W_edge, b_edge, W_node, b_node, W1, b1, W2, b2):
    full = lambda shape: pl.BlockSpec(shape, lambda i: tuple(0 for _ in shape))
    out = pl.pallas_call(
        _body,
        grid=(_N // _TJ,),
        in_specs=[full((_N, _N)), full((_N, _N)), full((_DE, 1)),
                  full((_DN, _DE)), full((_DN, 1)), full((_DN, _DH)),
                  full((_DN, _DH)), full((1, _DH)), full((_DH, 1)),
                  full((1, 1))],
        out_specs=pl.BlockSpec((_N, _TJ), lambda i: (0, i)),
        out_shape=jax.ShapeDtypeStruct((_N, _N), jnp.float32),
        scratch_shapes=[pltpu.VMEM((_N, _DH), jnp.float32),
                        pltpu.VMEM((_N, _DH), jnp.float32)],
    )(A.reshape(_N, _N), B.reshape(_N, _N), W_edge.T, W_node.T,
      b_node.reshape(_DN, 1), W1[:_DN], W1[_DN:], b1.reshape(1, _DH),
      W2, b2.reshape(1, 1))
    return out
